# Initial kernel scaffold; baseline (speedup 1.0000x reference)
import jax
import jax.numpy as jnp
from jax.experimental import pallas as pl


def kernel(x, t, loc_table0, loc_table1, loc_table2, time_table0, time_table1):
    raise NotImplementedError("write your pallas kernel here")


# speedup vs baseline: 2.1981x; 2.1981x over previous
import functools

import jax
import jax.numpy as jnp
from jax import lax
from jax.experimental import pallas as pl
from jax.experimental.pallas import tpu as pltpu
from jax.experimental.pallas import tpu_sc as plsc

B, L = 16384, 50
N = B * L
D0, D1, D2, DT = 80, 32, 16, 16
DOUT = D0 + D1 + D2 + 2 * DT
C = 128

NC, NS, LANES = 2, 16, 16
NW = NC * NS
PER_W = N // NW
N_CHUNKS = PER_W // C

_mesh = plsc.VectorSubcoreMesh(
    core_axis_name="c", subcore_axis_name="s", num_cores=NC, num_subcores=NS
)


@functools.partial(
    pl.kernel,
    mesh=_mesh,
    compiler_params=pltpu.CompilerParams(
        needs_layout_passes=False, use_tc_tiling_on_sc=False
    ),
    out_type=jax.ShapeDtypeStruct((N * DOUT,), jnp.float32),
    scratch_types=[
        pltpu.VMEM((C * 3,), jnp.int32),
        pltpu.VMEM((C * 2,), jnp.int32),
        pltpu.VMEM((C,), jnp.int32),
        pltpu.VMEM((C,), jnp.int32),
        pltpu.VMEM((C,), jnp.int32),
        pltpu.VMEM((C,), jnp.int32),
        pltpu.VMEM((C,), jnp.int32),
        pltpu.VMEM((C, D0), jnp.float32),
        pltpu.VMEM((C, D1), jnp.float32),
        pltpu.VMEM((C, D2), jnp.float32),
        pltpu.VMEM((C, DT), jnp.float32),
        pltpu.VMEM((C, DT), jnp.float32),
        pltpu.VMEM((C * DOUT,), jnp.float32),
        pltpu.SemaphoreType.DMA,
    ],
)
def _emb_kernel(x_hbm, t_hbm, l0, l1, l2, tt0, tt1, out_hbm,
                xbuf, tbuf, i0, i1, i2, it0, it1,
                b0, b1, b2, bt0, bt1, cat, sem):
    wid = lax.axis_index("s") * NC + lax.axis_index("c")
    wbase = wid * PER_W

    def chunk_body(c, carry):
        base = wbase + c * C
        pltpu.sync_copy(x_hbm.at[pl.ds(base * 3, C * 3)], xbuf)
        pltpu.sync_copy(t_hbm.at[pl.ds(base * 2, C * 2)], tbuf)

        def extract(i, carry2):
            lanes = lax.iota(jnp.int32, LANES) + i * LANES
            i0[pl.ds(i * LANES, LANES)] = plsc.load_gather(xbuf, [lanes * 3])
            i1[pl.ds(i * LANES, LANES)] = plsc.load_gather(xbuf, [lanes * 3 + 1])
            i2[pl.ds(i * LANES, LANES)] = plsc.load_gather(xbuf, [lanes * 3 + 2])
            it0[pl.ds(i * LANES, LANES)] = plsc.load_gather(tbuf, [lanes * 2])
            it1[pl.ds(i * LANES, LANES)] = plsc.load_gather(tbuf, [lanes * 2 + 1])
            return carry2

        lax.fori_loop(0, C // LANES, extract, 0)

        cps = [
            pltpu.async_copy(l0.at[i0], b0, sem),
            pltpu.async_copy(l1.at[i1], b1, sem),
            pltpu.async_copy(l2.at[i2], b2, sem),
            pltpu.async_copy(tt0.at[it0], bt0, sem),
            pltpu.async_copy(tt1.at[it1], bt1, sem),
        ]
        for cp in cps:
            cp.wait()

        def assemble(r, carry2):
            o = r * DOUT
            for j in range(D0 // LANES):
                cat[pl.ds(o + j * LANES, LANES)] = b0[r, pl.ds(j * LANES, LANES)]
            for j in range(D1 // LANES):
                cat[pl.ds(o + D0 + j * LANES, LANES)] = b1[r, pl.ds(j * LANES, LANES)]
            cat[pl.ds(o + D0 + D1, LANES)] = b2[r, pl.ds(0, LANES)]
            cat[pl.ds(o + D0 + D1 + D2, LANES)] = bt0[r, pl.ds(0, LANES)]
            cat[pl.ds(o + D0 + D1 + D2 + DT, LANES)] = bt1[r, pl.ds(0, LANES)]
            return carry2

        lax.fori_loop(0, C, assemble, 0)

        pltpu.sync_copy(cat, out_hbm.at[pl.ds(base * DOUT, C * DOUT)])
        return carry

    lax.fori_loop(0, N_CHUNKS, chunk_body, 0)


def kernel(x, t, loc_table0, loc_table1, loc_table2, time_table0, time_table1):
    out = _emb_kernel(
        x.reshape(-1), t.reshape(-1),
        loc_table0, loc_table1, loc_table2, time_table0, time_table1,
    )
    return out.reshape(B, L, DOUT)


# speedup vs baseline: 2.8592x; 1.3008x over previous
import functools

import jax
import jax.numpy as jnp
from jax import lax
from jax.experimental import pallas as pl
from jax.experimental.pallas import tpu as pltpu
from jax.experimental.pallas import tpu_sc as plsc

B, L = 16384, 50
N = B * L
D0, D1, D2, DT = 80, 32, 16, 16
DOUT = D0 + D1 + D2 + 2 * DT
DPAD = 128

NC, NS, LANES = 2, 16, 16
NW = NC * NS
BROWS_W = B // NW
BCHUNK = 4
C = BCHUNK * L
N_CHUNKS = BROWS_W // BCHUNK
NVEC = (C + LANES - 1) // LANES
CPAD = NVEC * LANES

_mesh = plsc.VectorSubcoreMesh(
    core_axis_name="c", subcore_axis_name="s", num_cores=NC, num_subcores=NS
)


@functools.partial(
    pl.kernel,
    mesh=_mesh,
    compiler_params=pltpu.CompilerParams(needs_layout_passes=False),
    out_type=jax.ShapeDtypeStruct((B, L, DOUT), jnp.float32),
    scratch_types=[
        pltpu.VMEM((C * 3,), jnp.int32),
        pltpu.VMEM((C * 2,), jnp.int32),
        pltpu.VMEM((CPAD,), jnp.int32),
        pltpu.VMEM((CPAD,), jnp.int32),
        pltpu.VMEM((CPAD,), jnp.int32),
        pltpu.VMEM((C, DPAD), jnp.float32),
        pltpu.VMEM((C, DPAD), jnp.float32),
        pltpu.VMEM((C, DPAD), jnp.float32),
        pltpu.VMEM((24, DT), jnp.float32),
        pltpu.VMEM((7, DT), jnp.float32),
        pltpu.VMEM((BCHUNK // 2, L, DOUT), jnp.float32),
        pltpu.SemaphoreType.DMA,
    ],
)
def _emb_kernel(x_hbm, t_hbm, l0, l1, l2, tt0, tt1, out_hbm,
                xbuf, tbuf, i0, i1, i2, b0, b1, b2, t0v, t1v, cat, sem):
    wid = lax.axis_index("s") * NC + lax.axis_index("c")
    wrow = wid * BROWS_W

    pltpu.sync_copy(tt0, t0v)
    pltpu.sync_copy(tt1, t1v)
    HC = C // 2
    HV = (HC + LANES - 1) // LANES

    def chunk_body(c, carry):
        brow = wrow + c * BCHUNK
        base = brow * L
        pltpu.sync_copy(x_hbm.at[pl.ds(base * 3, C * 3)], xbuf)
        pltpu.sync_copy(t_hbm.at[pl.ds(base * 2, C * 2)], tbuf)

        def extract(i, carry2):
            p = jnp.minimum(lax.iota(jnp.int32, LANES) + i * LANES, C - 1)
            i0[pl.ds(i * LANES, LANES)] = plsc.load_gather(xbuf, [p * 3])
            i1[pl.ds(i * LANES, LANES)] = plsc.load_gather(xbuf, [p * 3 + 1])
            i2[pl.ds(i * LANES, LANES)] = plsc.load_gather(xbuf, [p * 3 + 2])
            return carry2

        lax.fori_loop(0, NVEC, extract, 0)

        cps = []
        for off, nrow in ((0, 80), (80, 80), (160, 40)):
            rows = pl.ds(off, nrow)
            cps.append(pltpu.async_copy(l0.at[i0.at[rows]], b0.at[rows], sem))
            cps.append(pltpu.async_copy(l1.at[i1.at[rows]], b1.at[rows], sem))
            cps.append(pltpu.async_copy(l2.at[i2.at[rows]], b2.at[rows], sem))

        for cp in cps:
            cp.wait()

        for h in range(2):
            hbase = h * HC

            def timestep(i, carry2):
                p = jnp.minimum(lax.iota(jnp.int32, LANES) + i * LANES, HC - 1)
                bi = p // L
                li = p - bi * L
                pg = p + hbase
                t0 = plsc.load_gather(tbuf, [pg * 2])
                t1 = plsc.load_gather(tbuf, [pg * 2 + 1])
                for j in range(DT):
                    jv = jnp.full((LANES,), j, jnp.int32)
                    v0 = plsc.load_gather(t0v, [t0, jv])
                    plsc.store_scatter(cat, [bi, li, jv + (D0 + D1 + D2)], v0)
                    v1 = plsc.load_gather(t1v, [t1, jv])
                    plsc.store_scatter(cat, [bi, li, jv + (D0 + D1 + D2 + DT)], v1)
                return carry2

            lax.fori_loop(0, HV, timestep, 0)

            def assemble(r, carry2):
                bi = r // L
                li = r - bi * L
                g = r + hbase
                for j in range(D0 // LANES):
                    cat[bi, li, pl.ds(j * LANES, LANES)] = b0[g, pl.ds(j * LANES, LANES)]
                for j in range(D1 // LANES):
                    cat[bi, li, pl.ds(D0 + j * LANES, LANES)] = b1[g, pl.ds(j * LANES, LANES)]
                cat[bi, li, pl.ds(D0 + D1, LANES)] = b2[g, pl.ds(0, LANES)]
                return carry2

            lax.fori_loop(0, HC, assemble, 0)

            pltpu.sync_copy(cat, out_hbm.at[pl.ds(brow + h * (BCHUNK // 2), BCHUNK // 2)])
        return carry

    lax.fori_loop(0, N_CHUNKS, chunk_body, 0)


def kernel(x, t, loc_table0, loc_table1, loc_table2, time_table0, time_table1):
    l0p = jnp.pad(loc_table0, ((0, 0), (0, DPAD - D0)))
    l1p = jnp.pad(loc_table1, ((0, 0), (0, DPAD - D1)))
    l2p = jnp.pad(loc_table2, ((0, 0), (0, DPAD - D2)))
    return _emb_kernel(
        x.reshape(-1), t.reshape(-1), l0p, l1p, l2p, time_table0, time_table1
    )
